# Initial kernel scaffold; baseline (speedup 1.0000x reference)
import jax
import jax.numpy as jnp
from jax.experimental import pallas as pl


def kernel(out_xy, out_wh, out_confidence, out_class, target):
    raise NotImplementedError("write your pallas kernel here")


# speedup vs baseline: 6.6379x; 6.6379x over previous
import jax
import jax.numpy as jnp
from jax import lax
from jax.experimental import pallas as pl
from jax.experimental.pallas import tpu as pltpu
from jax.experimental.pallas import tpu_sc as plsc

POS_W, NOOBJ_W, OBJ_W, CLASS_W = 5.0, 0.5, 1.0, 1.0
B, A, H, W, C, M = 32, 3, 64, 64, 80, 100
MP = 128
HW = H * W
AHW = A * HW
NRC = B * AHW * C // 128
N1 = float(B * AHW)
N2 = float(B * AHW * 2)
NCL = float(B * AHW * C)
RNE_MAGIC = 12582912.0
LN2 = 0.6931471805599453


def _iota16():
    return lax.iota(jnp.int32, 16)


def _ln(v):
    v = jnp.maximum(v, 1e-30)
    bits = plsc.bitcast(v, jnp.int32)
    e = jnp.right_shift(bits, 23) - 127
    mbits = jnp.bitwise_or(jnp.bitwise_and(bits, 0x007FFFFF), 0x3F800000)
    m = plsc.bitcast(mbits, jnp.float32)
    big = m > 1.4142135
    m = jnp.where(big, m * 0.5, m)
    ef = (e + big.astype(jnp.int32)).astype(jnp.float32)
    z = (m - 1.0) / (m + 1.0)
    z2 = z * z
    p = jnp.float32(1.0 / 9.0)
    p = p * z2 + jnp.float32(1.0 / 7.0)
    p = p * z2 + jnp.float32(0.2)
    p = p * z2 + jnp.float32(1.0 / 3.0)
    p = p * z2 + jnp.float32(1.0)
    return 2.0 * z * p + ef * jnp.float32(LN2)


def _sqrt(v):
    v = jnp.maximum(v, 1e-24)
    y = plsc.bitcast(0x5F3759DF - jnp.right_shift(plsc.bitcast(v, jnp.int32), 1),
                     jnp.float32)
    for _ in range(3):
        y = y * (1.5 - 0.5 * v * y * y)
    return v * y


def _sc_body(targ_h, owh_h, oxy_h, cfr_h, cls_h, out_h,
             targ_v, ridx1, ridx2, buf1, buf2, x_v, y_v,
             aw_v, ah_v, iou_v, occ_v, kcell_v, q_v, astar_v,
             owsel_v, ohsel_v, vvalid_v, w_v, slot_v, coloff_v,
             ox_v, oy_v, tab_cell, tab_conf, tab_kc,
             dense_v, res_v, sem1, sem2, sem_d):
    iot = _iota16()
    z16 = jnp.zeros((16,), jnp.int32)
    b = lax.axis_index("s") * 2 + lax.axis_index("c")
    dcp = pltpu.async_copy(cfr_h.at[pl.ds(b * 96, 96)], dense_v, sem_d)
    pltpu.sync_copy(targ_h.at[b], targ_v)

    for ci in range(8):
        sl = pl.ds(ci * 16, 16)
        tx = targ_v[pl.ds(0 * MP + ci * 16, 16)]
        ty = targ_v[pl.ds(1 * MP + ci * 16, 16)]
        xf = (tx * jnp.float32(W) - 0.5 + RNE_MAGIC) - RNE_MAGIC
        yf = (ty * jnp.float32(H) - 0.5 + RNE_MAGIC) - RNE_MAGIC
        xi = jnp.clip(xf.astype(jnp.int32), 0, W - 1)
        yi = jnp.clip(yf.astype(jnp.int32), 0, H - 1)
        x_v[sl] = xi
        y_v[sl] = yi
        q_v[sl] = yi * W + xi
        ridx1[sl] = (b * 3 + 0) * H + yi
        ridx2[sl] = (b * 3 + 1) * H + yi
    g0 = pltpu.async_copy(owh_h.at[ridx1], buf1, sem1)
    g1 = pltpu.async_copy(owh_h.at[ridx2], buf2, sem2)

    carry = jnp.int32(0)
    for ci in range(8):
        sl = pl.ds(ci * 16, 16)
        tx = targ_v[pl.ds(0 * MP + ci * 16, 16)]
        neg = (tx < 0.0).astype(jnp.int32)
        cs = jnp.cumsum(neg)
        vvalid_v[sl] = ((cs + carry) == 0).astype(jnp.int32)
        carry = carry + jnp.sum(neg)

    neg1 = jnp.full((16,), -1, jnp.int32)

    def _clr(tab, n):
        def body(i, c):
            plsc.store_scatter(tab, [i * 16 + iot], neg1)
            return c
        lax.fori_loop(0, n // 16, body, jnp.int32(0))

    _clr(tab_cell, AHW)
    _clr(tab_conf, HW)
    _clr(tab_kc, MP * C)

    g0.wait()
    for ci in range(8):
        sl = pl.ds(ci * 16, 16)
        midx = ci * 16 + iot
        x2 = x_v[sl] * 2
        aw_v[pl.ds(0 * MP + ci * 16, 16)] = plsc.load_gather(buf1, [midx, x2])
        ah_v[pl.ds(0 * MP + ci * 16, 16)] = plsc.load_gather(buf1, [midx, x2 + 1])
    for ci in range(8):
        sl = pl.ds(ci * 16, 16)
        ridx1[sl] = (b * 3 + 2) * H + y_v[sl]
    g2 = pltpu.async_copy(owh_h.at[ridx1], buf1, sem1)
    g1.wait()
    for ci in range(8):
        sl = pl.ds(ci * 16, 16)
        midx = ci * 16 + iot
        x2 = x_v[sl] * 2
        aw_v[pl.ds(1 * MP + ci * 16, 16)] = plsc.load_gather(buf2, [midx, x2])
        ah_v[pl.ds(1 * MP + ci * 16, 16)] = plsc.load_gather(buf2, [midx, x2 + 1])
    g2.wait()
    for ci in range(8):
        sl = pl.ds(ci * 16, 16)
        midx = ci * 16 + iot
        x2 = x_v[sl] * 2
        aw_v[pl.ds(2 * MP + ci * 16, 16)] = plsc.load_gather(buf1, [midx, x2])
        ah_v[pl.ds(2 * MP + ci * 16, 16)] = plsc.load_gather(buf1, [midx, x2 + 1])

    for ci in range(8):
        sl = pl.ds(ci * 16, 16)
        tw = targ_v[pl.ds(2 * MP + ci * 16, 16)]
        th = targ_v[pl.ds(3 * MP + ci * 16, 16)]
        aw_a, ah_a, iou_a = [], [], []
        for aa in range(3):
            aw = aw_v[pl.ds(aa * MP + ci * 16, 16)]
            ah = ah_v[pl.ds(aa * MP + ci * 16, 16)]
            inter = jnp.minimum(tw, aw) * jnp.minimum(th, ah)
            union = tw * th + aw * ah - inter
            iou = inter / (union + 1e-9)
            iou_v[pl.ds(aa * MP + ci * 16, 16)] = iou
            aw_a.append(aw)
            ah_a.append(ah)
            iou_a.append(iou)
        best = iou_a[0]
        a = z16
        a = jnp.where(iou_a[1] > best, 1, a)
        best = jnp.maximum(best, iou_a[1])
        a = jnp.where(iou_a[2] > best, 2, a)
        astar_v[sl] = a
        kcell_v[sl] = a * HW + q_v[sl]
        owsel_v[sl] = jnp.where(a == 0, aw_a[0], jnp.where(a == 1, aw_a[1], aw_a[2]))
        ohsel_v[sl] = jnp.where(a == 0, ah_a[0], jnp.where(a == 1, ah_a[1], ah_a[2]))

    for ci in range(8):
        sl = pl.ds(ci * 16, 16)
        kc = kcell_v[sl]
        q = q_v[sl]
        vm = vvalid_v[sl] > 0
        mvec = ci * 16 + iot
        for l in range(16):
            msk = jnp.logical_and(vm, iot == l)
            plsc.store_scatter(tab_cell, [kc], mvec, mask=msk)
            plsc.store_scatter(tab_conf, [q], mvec, mask=msk)

    for ci in range(8):
        sl = pl.ds(ci * 16, 16)
        ridx1[sl] = (b * 3 + astar_v[sl]) * H + y_v[sl]
        ridx2[sl] = ((b * 3 + 0) * H + y_v[sl]) >> 1
    gxy = pltpu.async_copy(oxy_h.at[ridx1], buf1, sem1)
    gcf = pltpu.async_copy(cfr_h.at[ridx2], buf2, sem2)

    for ci in range(8):
        sl = pl.ds(ci * 16, 16)
        kc = kcell_v[sl]
        vm = vvalid_v[sl] > 0
        rep = plsc.load_gather(tab_cell, [kc])
        cl = jnp.clip(targ_v[pl.ds(4 * MP + ci * 16, 16)].astype(jnp.int32), 0, C - 1)
        slot = jnp.maximum(rep * C + cl, 0)
        slot_v[sl] = slot
        plsc.store_scatter(tab_kc, [slot], ci * 16 + iot, mask=vm)

    gxy.wait()
    for ci in range(8):
        sl = pl.ds(ci * 16, 16)
        midx = ci * 16 + iot
        x2 = x_v[sl] * 2
        ox_v[sl] = plsc.load_gather(buf1, [midx, x2])
        oy_v[sl] = plsc.load_gather(buf1, [midx, x2 + 1])
    for ci in range(8):
        sl = pl.ds(ci * 16, 16)
        ridx1[sl] = ((b * 3 + 1) * H + y_v[sl]) >> 1
    gc1 = pltpu.async_copy(cfr_h.at[ridx1], buf1, sem1)
    gcf.wait()
    for ci in range(8):
        sl = pl.ds(ci * 16, 16)
        midx = ci * 16 + iot
        ccol = jnp.bitwise_and(y_v[sl], 1) * 64 + x_v[sl]
        occ_v[pl.ds(0 * MP + ci * 16, 16)] = plsc.load_gather(buf2, [midx, ccol])
    for ci in range(8):
        sl = pl.ds(ci * 16, 16)
        ridx2[sl] = ((b * 3 + 2) * H + y_v[sl]) >> 1
    gc2 = pltpu.async_copy(cfr_h.at[ridx2], buf2, sem2)
    gc1.wait()
    for ci in range(8):
        sl = pl.ds(ci * 16, 16)
        midx = ci * 16 + iot
        ccol = jnp.bitwise_and(y_v[sl], 1) * 64 + x_v[sl]
        occ_v[pl.ds(1 * MP + ci * 16, 16)] = plsc.load_gather(buf1, [midx, ccol])
    gc2.wait()
    for ci in range(8):
        sl = pl.ds(ci * 16, 16)
        midx = ci * 16 + iot
        ccol = jnp.bitwise_and(y_v[sl], 1) * 64 + x_v[sl]
        occ_v[pl.ds(2 * MP + ci * 16, 16)] = plsc.load_gather(buf2, [midx, ccol])

    for ci in range(8):
        sl = pl.ds(ci * 16, 16)
        t = (b * AHW + kcell_v[sl]) * C
        r = jnp.right_shift(t, 7)
        coloff_v[sl] = jnp.bitwise_and(t, 127)
        ridx1[sl] = r
        ridx2[sl] = jnp.minimum(r + 1, NRC - 1)
    gk1 = pltpu.async_copy(cls_h.at[ridx1], buf1, sem1)
    gk2 = pltpu.async_copy(cls_h.at[ridx2], buf2, sem2)
    gk1.wait()
    gk2.wait()

    zf = jnp.zeros((16,), jnp.float32)
    t1a = t2a = t3a = t4a = t5a = t7a = zf
    for ci in range(8):
        sl = pl.ds(ci * 16, 16)
        midx = ci * 16 + iot
        mvec = midx
        kc = kcell_v[sl]
        q = q_v[sl]
        a = astar_v[sl]
        vm = vvalid_v[sl] > 0
        wl = jnp.logical_and(plsc.load_gather(tab_cell, [kc]) == mvec, vm)
        ilc = jnp.logical_and(plsc.load_gather(tab_conf, [q]) == mvec, vm)
        w_v[sl] = wl.astype(jnp.float32)
        occ = [occ_v[pl.ds(aa * MP + ci * 16, 16)] for aa in range(3)]
        for aa in range(3):
            objf = plsc.load_gather(tab_cell, [aa * HW + q]) >= 0
            iouv = iou_v[pl.ds(aa * MP + ci * 16, 16)]
            term = (occ[aa] - iouv) * (occ[aa] - iouv) - occ[aa] * occ[aa]
            t5a = t5a + jnp.where(jnp.logical_and(ilc, jnp.logical_not(objf)), term, 0.0)
        m2 = jnp.maximum(plsc.load_gather(tab_conf, [q]), 0)
        ioug = plsc.load_gather(iou_v, [a * MP + m2])
        oc_sel = jnp.where(a == 0, occ[0], jnp.where(a == 1, occ[1], occ[2]))
        t3a = t3a + jnp.where(wl, (oc_sel - ioug) * (oc_sel - ioug), 0.0)
        t4a = t4a + jnp.where(wl, oc_sel * oc_sel, 0.0)
        tx = targ_v[pl.ds(0 * MP + ci * 16, 16)]
        ty = targ_v[pl.ds(1 * MP + ci * 16, 16)]
        ox = ox_v[sl]
        oy = oy_v[sl]
        t1a = t1a + jnp.where(wl, (ox - tx) * (ox - tx) + (oy - ty) * (oy - ty), 0.0)
        tw = targ_v[pl.ds(2 * MP + ci * 16, 16)]
        th = targ_v[pl.ds(3 * MP + ci * 16, 16)]
        dw = _sqrt(owsel_v[sl]) - _sqrt(tw)
        dh = _sqrt(ohsel_v[sl]) - _sqrt(th)
        t2a = t2a + jnp.where(wl, dw * dw + dh * dh, 0.0)
        iwk = jnp.logical_and(plsc.load_gather(tab_kc, [slot_v[sl]]) == mvec, vm)
        cl = jnp.clip(targ_v[pl.ds(4 * MP + ci * 16, 16)].astype(jnp.int32), 0, C - 1)
        col = coloff_v[sl] + cl
        pmc = jnp.where(col < 128,
                        plsc.load_gather(buf1, [midx, jnp.minimum(col, 127)]),
                        plsc.load_gather(buf2, [midx, jnp.maximum(col - 128, 0)]))
        t7a = t7a + jnp.where(iwk, _ln(1.0 - pmc) - _ln(pmc), 0.0)

    def _t6(i, acc):
        row = i // 5
        cb = (i - row * 5) * 16
        rowv = z16 + row
        wv = plsc.load_gather(w_v, [rowv])
        off = plsc.load_gather(coloff_v, [rowv])
        col = off + cb + iot
        pv = jnp.where(col < 128,
                       plsc.load_gather(buf1, [rowv, jnp.minimum(col, 127)]),
                       plsc.load_gather(buf2, [rowv, jnp.maximum(col - 128, 0)]))
        return acc - wv * _ln(1.0 - pv)

    t6a = lax.fori_loop(0, MP * 5, _t6, zf)

    dcp.wait()

    def _dns(i, acc):
        rowv = z16 + i
        for j in range(8):
            xv = plsc.load_gather(dense_v, [rowv, j * 16 + iot])
            acc = acc + xv * xv
        return acc

    sdense = lax.fori_loop(0, 96, _dns, zf)

    res = (jnp.float32(POS_W / N2) * (t1a + t2a)
           + jnp.float32(OBJ_W / N1) * t3a
           + jnp.float32(NOOBJ_W / N1) * (sdense - t4a + t5a)
           + jnp.float32(CLASS_W / NCL) * (t6a + t7a))
    res_v[...] = res
    pltpu.sync_copy(res_v, out_h.at[b])


_SCRATCH = [
    pltpu.VMEM((5 * MP,), jnp.float32),
    pltpu.VMEM((MP,), jnp.int32),
    pltpu.VMEM((MP,), jnp.int32),
    pltpu.VMEM((MP, 128), jnp.float32),
    pltpu.VMEM((MP, 128), jnp.float32),
    pltpu.VMEM((MP,), jnp.int32),
    pltpu.VMEM((MP,), jnp.int32),
    pltpu.VMEM((3 * MP,), jnp.float32),
    pltpu.VMEM((3 * MP,), jnp.float32),
    pltpu.VMEM((3 * MP,), jnp.float32),
    pltpu.VMEM((3 * MP,), jnp.float32),
    pltpu.VMEM((MP,), jnp.int32),
    pltpu.VMEM((MP,), jnp.int32),
    pltpu.VMEM((MP,), jnp.int32),
    pltpu.VMEM((MP,), jnp.float32),
    pltpu.VMEM((MP,), jnp.float32),
    pltpu.VMEM((MP,), jnp.int32),
    pltpu.VMEM((MP,), jnp.float32),
    pltpu.VMEM((MP,), jnp.int32),
    pltpu.VMEM((MP,), jnp.int32),
    pltpu.VMEM((MP,), jnp.float32),
    pltpu.VMEM((MP,), jnp.float32),
    pltpu.VMEM((AHW,), jnp.int32),
    pltpu.VMEM((HW,), jnp.int32),
    pltpu.VMEM((MP * C,), jnp.int32),
    pltpu.VMEM((96, 128), jnp.float32),
    pltpu.VMEM((16,), jnp.float32),
    pltpu.SemaphoreType.DMA,
    pltpu.SemaphoreType.DMA,
    pltpu.SemaphoreType.DMA,
]

_sc_call = pl.kernel(
    _sc_body,
    out_type=jax.ShapeDtypeStruct((B, 16), jnp.float32),
    mesh=plsc.VectorSubcoreMesh(core_axis_name="c", subcore_axis_name="s",
                                num_cores=2, num_subcores=16),
    scratch_types=_SCRATCH,
    compiler_params=pltpu.CompilerParams(needs_layout_passes=False),
)


def kernel(out_xy, out_wh, out_confidence, out_class, target):
    tx = jnp.pad(target[..., 0], ((0, 0), (0, MP - M)), constant_values=-1.0)
    ty = jnp.pad(target[..., 1], ((0, 0), (0, MP - M)))
    tw = jnp.pad(target[..., 2], ((0, 0), (0, MP - M)))
    th = jnp.pad(target[..., 3], ((0, 0), (0, MP - M)))
    tc = jnp.pad(target[..., 4], ((0, 0), (0, MP - M)))
    targ_slab = jnp.concatenate([tx, ty, tw, th, tc], axis=1)
    partials = _sc_call(targ_slab,
                        out_wh.reshape(B * A * H, 128),
                        out_xy.reshape(B * A * H, 128),
                        out_confidence.reshape(B * AHW // 128, 128),
                        out_class.reshape(NRC, 128))
    return jnp.sum(partials)
